# Initial kernel scaffold; baseline (speedup 1.0000x reference)
import jax
import jax.numpy as jnp
from jax.experimental import pallas as pl


def kernel(hashes, embedding, pe, W, b):
    raise NotImplementedError("write your pallas kernel here")


# speedup vs baseline: 13.0457x; 13.0457x over previous
import functools

import jax
import jax.numpy as jnp
from jax import lax
from jax.experimental import pallas as pl
from jax.experimental.pallas import tpu as pltpu
from jax.experimental.pallas import tpu_sc as plsc

NUM_PARTS = 256
NUM_K = 2047
STRIDE = NUM_K + 1
VAE_DIMS = 32
OUT_DIMS = 32
BATCH = 1024

NW = 32
ROWS_TOTAL = BATCH * NUM_PARTS
ROWS_PER_W = ROWS_TOTAL // NW
IDX_ROWS = ROWS_PER_W // 128
CHUNK = 1024
NCHUNK = ROWS_PER_W // CHUNK
GPC = CHUNK // 128


def _sc_gather_body(h2d, emb, mu_out, idx2d, rows0, rows1, gsem, wsem):
    cid = lax.axis_index("c")
    sid = lax.axis_index("s")
    wid = sid * 2 + cid
    pltpu.sync_copy(h2d.at[pl.ds(wid * IDX_ROWS, IDX_ROWS)], idx2d)

    lane = lax.iota(jnp.int32, 16)

    def add_offs(j, carry):
        row = j // 8
        col = (j % 8) * 16
        offs = ((j % 16) * 16 + lane) * STRIDE
        v = idx2d[row, pl.ds(col, 16)]
        idx2d[row, pl.ds(col, 16)] = v + offs
        return carry

    lax.fori_loop(0, IDX_ROWS * 8, add_offs, 0)

    rows = [rows0, rows1]
    base = wid * ROWS_PER_W
    wb_handles = [None, None]
    for c in range(NCHUNK):
        buf = rows[c % 2]
        if wb_handles[c % 2] is not None:
            wb_handles[c % 2].wait()
        ghandles = []
        for k in range(GPC):
            ghandles.append(pltpu.async_copy(
                emb.at[idx2d.at[c * GPC + k]],
                buf.at[pl.ds(k * 128, 128)],
                gsem))
        for h in ghandles:
            h.wait()
        wb_handles[c % 2] = pltpu.async_copy(
            buf, mu_out.at[pl.ds(base + c * CHUNK, CHUNK)], wsem)
    for h in wb_handles:
        if h is not None:
            h.wait()


def _sc_gather(h2d, emb):
    mesh = plsc.VectorSubcoreMesh(core_axis_name="c", subcore_axis_name="s")
    f = functools.partial(
        pl.kernel,
        mesh=mesh,
        compiler_params=pltpu.CompilerParams(use_tc_tiling_on_sc=False),
        out_type=jax.ShapeDtypeStruct((ROWS_TOTAL, VAE_DIMS), jnp.float32),
        scratch_types=[
            pltpu.VMEM((IDX_ROWS, 128), jnp.int32),
            pltpu.VMEM((CHUNK, VAE_DIMS), jnp.float32),
            pltpu.VMEM((CHUNK, VAE_DIMS), jnp.float32),
            pltpu.SemaphoreType.DMA,
            pltpu.SemaphoreType.DMA,
        ],
    )(_sc_gather_body)
    return f(h2d, emb)


BLK_ROWS = 4096


def _tc_proj_body(mu_ref, w_ref, peb_ref, out_ref):
    mu = mu_ref[...]
    w = w_ref[...]
    acc = jnp.dot(mu, w, preferred_element_type=jnp.float32)
    peb = peb_ref[...]
    out_ref[...] = acc + jnp.tile(peb, (BLK_ROWS // NUM_PARTS, 1))


def _tc_proj(mu, w, peb):
    grid = (ROWS_TOTAL // BLK_ROWS,)
    return pl.pallas_call(
        _tc_proj_body,
        grid=grid,
        in_specs=[
            pl.BlockSpec((BLK_ROWS, VAE_DIMS), lambda i: (i, 0)),
            pl.BlockSpec((VAE_DIMS, OUT_DIMS), lambda i: (0, 0)),
            pl.BlockSpec((NUM_PARTS, OUT_DIMS), lambda i: (0, 0)),
        ],
        out_specs=pl.BlockSpec((BLK_ROWS, OUT_DIMS), lambda i: (i, 0)),
        out_shape=jax.ShapeDtypeStruct((ROWS_TOTAL, OUT_DIMS), jnp.float32),
    )(mu, w, peb)


def kernel(hashes, embedding, pe, W, b):
    B, P = hashes.shape
    h2d = hashes.reshape(-1, 128)
    mu = _sc_gather(h2d, embedding)
    peb = pe + b[None, :]
    out = _tc_proj(mu, W, peb)
    return out.reshape(B, P, OUT_DIMS)


# speedup vs baseline: 14.3329x; 1.0987x over previous
import functools

import jax
import jax.numpy as jnp
from jax import lax
from jax.experimental import pallas as pl
from jax.experimental.pallas import tpu as pltpu
from jax.experimental.pallas import tpu_sc as plsc

NUM_PARTS = 256
NUM_K = 2047
STRIDE = NUM_K + 1
VAE_DIMS = 32
OUT_DIMS = 32
BATCH = 1024

NW = 32
ROWS_TOTAL = BATCH * NUM_PARTS
ROWS_PER_W = ROWS_TOTAL // NW
IDX_ROWS = ROWS_PER_W // 128
CHUNK = 1024
NCHUNK = ROWS_PER_W // CHUNK
GPC = CHUNK // 128


def _sc_gather_body(h2d, emb, mu_out, idx2d, rows0, rows1, gsem, wsem):
    cid = lax.axis_index("c")
    sid = lax.axis_index("s")
    wid = sid * 2 + cid
    pltpu.sync_copy(h2d.at[pl.ds(wid * IDX_ROWS, IDX_ROWS)], idx2d)

    lane = lax.iota(jnp.int32, 16)

    def add_offs(j, carry):
        row = j // 8
        col = (j % 8) * 16
        offs = ((j % 16) * 16 + lane) * STRIDE
        v = idx2d[row, pl.ds(col, 16)]
        idx2d[row, pl.ds(col, 16)] = v + offs
        return carry

    lax.fori_loop(0, IDX_ROWS * 8, add_offs, 0)

    rows = [rows0, rows1]
    base = wid * ROWS_PER_W
    wb_handles = [None, None]
    for c in range(NCHUNK):
        buf = rows[c % 2]
        if wb_handles[c % 2] is not None:
            wb_handles[c % 2].wait()
        ghandles = []
        for k in range(GPC):
            ghandles.append(pltpu.async_copy(
                emb.at[idx2d.at[c * GPC + k]],
                buf.at[pl.ds(k * 128, 128)],
                gsem))
        for h in ghandles:
            h.wait()
        wb_handles[c % 2] = pltpu.async_copy(
            buf, mu_out.at[pl.ds(base + c * CHUNK, CHUNK)], wsem)
    for h in wb_handles:
        if h is not None:
            h.wait()


def _sc_gather(h2d, emb):
    mesh = plsc.VectorSubcoreMesh(core_axis_name="c", subcore_axis_name="s")
    f = functools.partial(
        pl.kernel,
        mesh=mesh,
        compiler_params=pltpu.CompilerParams(use_tc_tiling_on_sc=False),
        out_type=jax.ShapeDtypeStruct((ROWS_TOTAL, VAE_DIMS), jnp.float32),
        scratch_types=[
            pltpu.VMEM((IDX_ROWS, 128), jnp.int32),
            pltpu.VMEM((CHUNK, VAE_DIMS), jnp.float32),
            pltpu.VMEM((CHUNK, VAE_DIMS), jnp.float32),
            pltpu.SemaphoreType.DMA,
            pltpu.SemaphoreType.DMA,
        ],
    )(_sc_gather_body)
    return f(h2d, emb)


BLK_ROWS = 4096


BLK_B = BLK_ROWS // NUM_PARTS


def _tc_proj_body(mu_ref, w_ref, peb_ref, out_ref):
    mu = mu_ref[...]
    w = w_ref[...]
    acc = jnp.dot(mu, w, preferred_element_type=jnp.float32)
    peb = peb_ref[...]
    acc = acc + jnp.tile(peb, (BLK_B, 1))
    out_ref[...] = jnp.swapaxes(acc.reshape(BLK_B, NUM_PARTS, OUT_DIMS), 1, 2)


def _tc_proj(mu, w, peb):
    grid = (ROWS_TOTAL // BLK_ROWS,)
    return pl.pallas_call(
        _tc_proj_body,
        grid=grid,
        in_specs=[
            pl.BlockSpec((BLK_ROWS, VAE_DIMS), lambda i: (i, 0)),
            pl.BlockSpec((VAE_DIMS, OUT_DIMS), lambda i: (0, 0)),
            pl.BlockSpec((NUM_PARTS, OUT_DIMS), lambda i: (0, 0)),
        ],
        out_specs=pl.BlockSpec((BLK_B, OUT_DIMS, NUM_PARTS), lambda i: (i, 0, 0)),
        out_shape=jax.ShapeDtypeStruct((BATCH, OUT_DIMS, NUM_PARTS), jnp.float32),
    )(mu, w, peb)


def kernel(hashes, embedding, pe, W, b):
    B, P = hashes.shape
    h2d = hashes.reshape(-1, 128)
    mu = _sc_gather(h2d, embedding)
    peb = pe + b[None, :]
    out_t = _tc_proj(mu, W, peb)
    return jnp.swapaxes(out_t, 1, 2)
